# baseline (device time: 116642 ns/iter reference)
import jax
import jax.numpy as jnp
from jax import lax
from jax.experimental import pallas as pl
from jax.experimental.pallas import tpu as pltpu

N_DEV = 8
SQ = 256
SKV = 4096
HQ = 8
DH = 128
D = 1024
SCALE = 0.08838834764831843


def kernel(x, Wq, Wo, K_ext, V_ext):
    x2 = x.reshape(SQ, D).astype(jnp.bfloat16)
    wq = Wq.astype(jnp.bfloat16)
    wo = Wo.astype(jnp.bfloat16)
    k3 = jnp.transpose(K_ext.reshape(SKV, HQ, DH), (1, 0, 2)).astype(jnp.bfloat16)
    v3 = jnp.transpose(V_ext.reshape(SKV, HQ, DH), (1, 0, 2)).astype(jnp.bfloat16)

    def body(x_ref, wq_ref, wo_ref, k_ref, v_ref, out_ref,
             qbuf, osend, orecv, lsend, lrecv, attbuf,
             q_send_sems, q_recv_sems, o_send_sems, o_recv_sems,
             l_send_sems, l_recv_sems):
        my = lax.axis_index("i")

        barrier = pltpu.get_barrier_semaphore()
        for t in range(1, N_DEV):
            pl.semaphore_signal(
                barrier, inc=1,
                device_id=((my + t) % N_DEV,),
                device_id_type=pl.DeviceIdType.MESH,
            )
        pl.semaphore_wait(barrier, N_DEV - 1)

        q32 = jnp.dot(x_ref[...], wq_ref[...],
                      preferred_element_type=jnp.float32)
        qbuf[0, :, :] = (q32 * SCALE).astype(jnp.bfloat16)

        q_sends = []
        for t in range(1, N_DEV):
            rdma = pltpu.make_async_remote_copy(
                src_ref=qbuf.at[0],
                dst_ref=qbuf.at[t],
                send_sem=q_send_sems.at[t],
                recv_sem=q_recv_sems.at[t],
                device_id=((my + t) % N_DEV,),
                device_id_type=pl.DeviceIdType.MESH,
            )
            rdma.start()
            q_sends.append(rdma)

        def compute_block(j, o_dst, l_dst, oslot, lslot):
            for h in range(HQ):
                q_h = qbuf[j, :, h * DH:(h + 1) * DH]
                s = lax.dot_general(
                    q_h, k_ref[h],
                    (((1,), (1,)), ((), ())),
                    preferred_element_type=jnp.float32,
                )
                p = jnp.exp(s)
                l_dst[lslot, :, h:h + 1] = jnp.sum(p, axis=1, keepdims=True)
                o_h = lax.dot_general(
                    p.astype(jnp.bfloat16), v_ref[h],
                    (((1,), (0,)), ((), ())),
                    preferred_element_type=jnp.float32,
                )
                o_dst[oslot, :, h * DH:(h + 1) * DH] = o_h.astype(jnp.bfloat16)

        compute_block(0, orecv, lrecv, 0, 0)

        p_sends = []
        for j in range(1, N_DEV):
            q_sends[j - 1].wait_recv()
            slot = N_DEV - j
            compute_block(j, osend, lsend, slot, slot)
            owner = (my - j) % N_DEV
            ro = pltpu.make_async_remote_copy(
                src_ref=osend.at[slot],
                dst_ref=orecv.at[slot],
                send_sem=o_send_sems.at[slot],
                recv_sem=o_recv_sems.at[slot],
                device_id=(owner,),
                device_id_type=pl.DeviceIdType.MESH,
            )
            ro.start()
            rl = pltpu.make_async_remote_copy(
                src_ref=lsend.at[slot],
                dst_ref=lrecv.at[slot],
                send_sem=l_send_sems.at[slot],
                recv_sem=l_recv_sems.at[slot],
                device_id=(owner,),
                device_id_type=pl.DeviceIdType.MESH,
            )
            rl.start()
            p_sends.extend((ro, rl))

        acc_o = orecv[0].astype(jnp.float32)
        acc_l = lrecv[0]
        for j in range(1, N_DEV):
            ro_wait = pltpu.make_async_remote_copy(
                src_ref=osend.at[j], dst_ref=orecv.at[j],
                send_sem=o_send_sems.at[j], recv_sem=o_recv_sems.at[j],
                device_id=(my,), device_id_type=pl.DeviceIdType.MESH,
            )
            ro_wait.wait_recv()
            rl_wait = pltpu.make_async_remote_copy(
                src_ref=lsend.at[j], dst_ref=lrecv.at[j],
                send_sem=l_send_sems.at[j], recv_sem=l_recv_sems.at[j],
                device_id=(my,), device_id_type=pl.DeviceIdType.MESH,
            )
            rl_wait.wait_recv()
            acc_o = acc_o + orecv[j].astype(jnp.float32)
            acc_l = acc_l + lrecv[j]

        for h in range(HQ):
            att_h = acc_o[:, h * DH:(h + 1) * DH] / acc_l[:, h:h + 1]
            attbuf[:, h * DH:(h + 1) * DH] = att_h.astype(jnp.bfloat16)

        out_ref[...] = jnp.dot(attbuf[...], wo_ref[...],
                               preferred_element_type=jnp.float32)

        for rdma in q_sends + p_sends:
            rdma.wait_send()

    y = pl.pallas_call(
        body,
        out_shape=jax.ShapeDtypeStruct((SQ, D), jnp.float32),
        in_specs=[pl.BlockSpec(memory_space=pltpu.VMEM)] * 5,
        out_specs=pl.BlockSpec(memory_space=pltpu.VMEM),
        scratch_shapes=[
            pltpu.VMEM((N_DEV, SQ, D), jnp.bfloat16),
            pltpu.VMEM((N_DEV, SQ, D), jnp.bfloat16),
            pltpu.VMEM((N_DEV, SQ, D), jnp.bfloat16),
            pltpu.VMEM((N_DEV, SQ, HQ), jnp.float32),
            pltpu.VMEM((N_DEV, SQ, HQ), jnp.float32),
            pltpu.VMEM((SQ, D), jnp.bfloat16),
            pltpu.SemaphoreType.DMA((N_DEV,)),
            pltpu.SemaphoreType.DMA((N_DEV,)),
            pltpu.SemaphoreType.DMA((N_DEV,)),
            pltpu.SemaphoreType.DMA((N_DEV,)),
            pltpu.SemaphoreType.DMA((N_DEV,)),
            pltpu.SemaphoreType.DMA((N_DEV,)),
        ],
        compiler_params=pltpu.CompilerParams(collective_id=0),
    )(x2, wq, wo, k3, v3)
    return y.reshape(1, SQ, D)
